# baseline (speedup 1.0000x reference)
import functools
import math

import jax
import jax.numpy as jnp
from jax.experimental import pallas as pl
from jax.experimental.pallas import tpu as pltpu

_INV_SQRT2 = 1.0 / math.sqrt(2.0)
_LANES = 128


def _router_kernel(sub_n, nbuf, nfold, nsteps, x_hbm, w1_ref, b1_ref, w2_ref,
                   b2_ref, packets_ref, probs_ref, xbuf, dma_sems):
    i = pl.program_id(0)
    rows = sub_n * nfold

    def start_copy(step, slot):
        pltpu.make_async_copy(
            x_hbm.at[pl.ds(step * rows, rows), :],
            xbuf.at[slot],
            dma_sems.at[slot],
        ).start()

    @pl.when(i == 0)
    def _():
        for s in range(min(nbuf - 1, nsteps)):
            start_copy(s, s)

    refill = i + nbuf - 1

    @pl.when(refill < nsteps)
    def _():
        start_copy(jnp.minimum(refill, nsteps - 1), refill % nbuf)

    slot = jax.lax.rem(i, nbuf)
    pltpu.make_async_copy(
        x_hbm.at[pl.ds(i * rows, rows), :],
        xbuf.at[slot],
        dma_sems.at[slot],
    ).wait()

    h = None
    for k in range(nfold):
        xk = xbuf[slot, pl.Slice(k, sub_n, nfold), :]
        w1k = w1_ref[pl.ds(k * _LANES, _LANES), :]
        part = jnp.dot(xk, w1k, preferred_element_type=jnp.float32)
        h = part if h is None else h + part
    h = h + b1_ref[...]
    h = 0.5 * h * (1.0 + jax.lax.erf(h * _INV_SQRT2))
    logits = jnp.dot(h, w2_ref[...], preferred_element_type=jnp.float32)
    logits = logits + b2_ref[...]
    packets_ref[...] = jnp.argmax(
        logits, axis=-1, keepdims=True).astype(jnp.int32)
    m = jnp.max(logits, axis=-1, keepdims=True)
    e = jnp.exp(logits - m)
    probs_ref[...] = e / jnp.sum(e, axis=-1, keepdims=True)


@functools.partial(jax.jit, static_argnames=("sub_n", "nbuf"))
def kernel(x, W1, b1, W2, b2, sub_n: int = 2048, nbuf: int = 6):
    n, d = x.shape
    h_dim = W1.shape[1]
    p = W2.shape[1]
    nfold = d // _LANES
    nsteps = n // sub_n
    x_fold = x.reshape(n * nfold, _LANES)
    packets2d, probs = pl.pallas_call(
        functools.partial(_router_kernel, sub_n, nbuf, nfold, nsteps),
        grid=(nsteps,),
        in_specs=[
            pl.BlockSpec(memory_space=pltpu.MemorySpace.HBM),
            pl.BlockSpec((d, h_dim), lambda i: (0, 0)),
            pl.BlockSpec((h_dim,), lambda i: (0,)),
            pl.BlockSpec((h_dim, p), lambda i: (0, 0)),
            pl.BlockSpec((p,), lambda i: (0,)),
        ],
        out_specs=[
            pl.BlockSpec((sub_n, 1), lambda i: (i, 0)),
            pl.BlockSpec((sub_n, p), lambda i: (i, 0)),
        ],
        out_shape=[
            jax.ShapeDtypeStruct((n, 1), jnp.int32),
            jax.ShapeDtypeStruct((n, p), jnp.float32),
        ],
        scratch_shapes=[
            pltpu.VMEM((nbuf, sub_n * nfold, _LANES), jnp.float32),
            pltpu.SemaphoreType.DMA((nbuf,)),
        ],
        compiler_params=pltpu.CompilerParams(
            dimension_semantics=("arbitrary",),
        ),
    )(x_fold, W1, b1, W2, b2)
    return packets2d.reshape(n), probs


# speedup vs baseline: 2.2977x; 2.2977x over previous
import functools
import math

import jax
import jax.numpy as jnp
from jax.experimental import pallas as pl
from jax.experimental.pallas import tpu as pltpu

_INV_SQRT2 = 1.0 / math.sqrt(2.0)


def _router_block(ksplit, kw, *refs):
    x_refs = refs[:ksplit]
    w1_ref, b1_ref, w2_ref, b2_ref = refs[ksplit:ksplit + 4]
    packets_ref, probs_ref = refs[ksplit + 4:]
    h = None
    for k in range(ksplit):
        w1k = w1_ref[pl.ds(k * kw, kw), :]
        part = jnp.dot(x_refs[k][...], w1k, preferred_element_type=jnp.float32)
        h = part if h is None else h + part
    h = h + b1_ref[...]
    h = 0.5 * h * (1.0 + jax.lax.erf(h * _INV_SQRT2))
    logits = jnp.dot(h, w2_ref[...], preferred_element_type=jnp.float32)
    logits = logits + b2_ref[...]
    packets_ref[...] = jnp.argmax(
        logits, axis=-1, keepdims=True).astype(jnp.int32)
    m = jnp.max(logits, axis=-1, keepdims=True)
    e = jnp.exp(logits - m)
    probs_ref[...] = e / jnp.sum(e, axis=-1, keepdims=True)


@functools.partial(jax.jit, static_argnames=("block_n", "ksplit"))
def kernel(x, W1, b1, W2, b2, block_n: int = 2048, ksplit: int = 2):
    n, d = x.shape
    h_dim = W1.shape[1]
    p = W2.shape[1]
    kw = d // ksplit
    grid = (n // block_n,)
    in_specs = [
        pl.BlockSpec((block_n, kw), functools.partial(
            lambda i, k=0: (i, k), k=k))
        for k in range(ksplit)
    ] + [
        pl.BlockSpec((d, h_dim), lambda i: (0, 0)),
        pl.BlockSpec((h_dim,), lambda i: (0,)),
        pl.BlockSpec((h_dim, p), lambda i: (0, 0)),
        pl.BlockSpec((p,), lambda i: (0,)),
    ]
    packets2d, probs = pl.pallas_call(
        functools.partial(_router_block, ksplit, kw),
        grid=grid,
        in_specs=in_specs,
        out_specs=[
            pl.BlockSpec((block_n, 1), lambda i: (i, 0)),
            pl.BlockSpec((block_n, p), lambda i: (i, 0)),
        ],
        out_shape=[
            jax.ShapeDtypeStruct((n, 1), jnp.int32),
            jax.ShapeDtypeStruct((n, p), jnp.float32),
        ],
        compiler_params=pltpu.CompilerParams(
            dimension_semantics=("arbitrary",),
        ),
    )(*([x] * ksplit), W1, b1, W2, b2)
    return packets2d.reshape(n), probs


# speedup vs baseline: 3.9633x; 1.7249x over previous
import functools
import math

import jax
import jax.numpy as jnp
from jax.experimental import pallas as pl
from jax.experimental.pallas import tpu as pltpu

_INV_SQRT2 = 1.0 / math.sqrt(2.0)


def _router_block(x_ref, w1_ref, b1_ref, w2_ref, b2c_ref,
                  packets_ref, probs_ref):
    h = jnp.dot(x_ref[...], w1_ref[...], preferred_element_type=jnp.float32)
    h = h + b1_ref[...]
    h = 0.5 * h * (1.0 + jax.lax.erf(h * _INV_SQRT2))
    logits_t = jax.lax.dot_general(
        w2_ref[...], h, (((0,), (1,)), ((), ())),
        preferred_element_type=jnp.float32)
    logits_t = logits_t + b2c_ref[...]
    pcount = logits_t.shape[0]
    m = jnp.max(logits_t, axis=0, keepdims=True)
    row_idx = jax.lax.broadcasted_iota(jnp.int32, logits_t.shape, 0)
    cand = jnp.where(logits_t == m, row_idx, pcount)
    packets_ref[...] = jnp.min(cand, axis=0, keepdims=True)
    e = jnp.exp(logits_t - m)
    probs_ref[...] = e / jnp.sum(e, axis=0, keepdims=True)


@functools.partial(jax.jit, static_argnames=("block_n",))
def kernel(x, W1, b1, W2, b2, block_n: int = 2048):
    n, d = x.shape
    h_dim = W1.shape[1]
    p = W2.shape[1]
    b2c = b2.reshape(p, 1)
    packets_row, probs_t = pl.pallas_call(
        _router_block,
        grid=(n // block_n,),
        in_specs=[
            pl.BlockSpec((block_n, d), lambda i: (i, 0)),
            pl.BlockSpec((d, h_dim), lambda i: (0, 0)),
            pl.BlockSpec((h_dim,), lambda i: (0,)),
            pl.BlockSpec((h_dim, p), lambda i: (0, 0)),
            pl.BlockSpec((p, 1), lambda i: (0, 0)),
        ],
        out_specs=[
            pl.BlockSpec((1, block_n), lambda i: (0, i)),
            pl.BlockSpec((p, block_n), lambda i: (0, i)),
        ],
        out_shape=[
            jax.ShapeDtypeStruct((1, n), jnp.int32),
            jax.ShapeDtypeStruct((p, n), jnp.float32),
        ],
        compiler_params=pltpu.CompilerParams(
            dimension_semantics=("arbitrary",),
        ),
    )(x, W1, b1, W2, b2c)
    return packets_row.reshape(n), probs_t.T


# speedup vs baseline: 4.1277x; 1.0415x over previous
import functools
import math

import jax
import jax.numpy as jnp
from jax.experimental import pallas as pl
from jax.experimental.pallas import tpu as pltpu

_INV_SQRT2 = 1.0 / math.sqrt(2.0)


def _router_block(x_ref, w1_ref, b1_ref, w2_ref, b2c_ref,
                  packets_ref, probs_ref):
    h = jnp.dot(x_ref[...], w1_ref[...], preferred_element_type=jnp.float32)
    h = h + b1_ref[...]
    h = 0.5 * h * (1.0 + jax.lax.erf(h * _INV_SQRT2))
    logits_t = jax.lax.dot_general(
        w2_ref[...], h, (((0,), (1,)), ((), ())),
        preferred_element_type=jnp.float32)
    logits_t = logits_t + b2c_ref[...]
    pcount = logits_t.shape[0]
    m = jnp.max(logits_t, axis=0, keepdims=True)
    row_idx = jax.lax.broadcasted_iota(jnp.int32, logits_t.shape, 0)
    cand = jnp.where(logits_t == m, row_idx, pcount)
    packets_ref[...] = jnp.min(cand, axis=0, keepdims=True)
    e = jnp.exp(logits_t - m)
    probs_ref[...] = e / jnp.sum(e, axis=0, keepdims=True)


@functools.partial(jax.jit, static_argnames=("block_n",))
def kernel(x, W1, b1, W2, b2, block_n: int = 4096):
    n, d = x.shape
    h_dim = W1.shape[1]
    p = W2.shape[1]
    b2c = b2.reshape(p, 1)
    packets_row, probs_t = pl.pallas_call(
        _router_block,
        grid=(n // block_n,),
        in_specs=[
            pl.BlockSpec((block_n, d), lambda i: (i, 0)),
            pl.BlockSpec((d, h_dim), lambda i: (0, 0)),
            pl.BlockSpec((h_dim,), lambda i: (0,)),
            pl.BlockSpec((h_dim, p), lambda i: (0, 0)),
            pl.BlockSpec((p, 1), lambda i: (0, 0)),
        ],
        out_specs=[
            pl.BlockSpec((1, block_n), lambda i: (0, i)),
            pl.BlockSpec((p, block_n), lambda i: (0, i)),
        ],
        out_shape=[
            jax.ShapeDtypeStruct((1, n), jnp.int32),
            jax.ShapeDtypeStruct((p, n), jnp.float32),
        ],
        compiler_params=pltpu.CompilerParams(
            dimension_semantics=("arbitrary",),
        ),
    )(x, W1, b1, W2, b2c)
    return packets_row.reshape(n), probs_t.T
